# baseline (speedup 1.0000x reference)
import functools

import jax
import jax.numpy as jnp
from jax.experimental import pallas as pl
from jax.experimental.pallas import tpu as pltpu

_OFF = 8
_CB = 64
_NBL = 128
_KW = 256


def _rb_kernel(x_ref, w1_ref, b1_ref, w2_ref, b2_ref, o_ref,
               scr, scr2, outf, *, H, Bt, S, NB):
    Mp = (Bt - 1) * S + H
    Wl = NB * _NBL

    scr[...] = jnp.zeros_like(scr)
    scr2[...] = jnp.zeros_like(scr2)

    for b in range(Bt):
        lo = _OFF + b * S
        scr[lo:lo + H, _CB:_CB + Wl] = x_ref[b]

    for j in range(NB):
        cw = j * _NBL
        a = jnp.dot(scr[_OFF - 1:_OFF - 1 + Mp, cw:cw + _KW], w1_ref[0, j],
                    preferred_element_type=jnp.float32)
        a += jnp.dot(scr[_OFF:_OFF + Mp, cw:cw + _KW], w1_ref[1, j],
                     preferred_element_type=jnp.float32)
        a += jnp.dot(scr[_OFF + 1:_OFF + 1 + Mp, cw:cw + _KW], w1_ref[2, j],
                     preferred_element_type=jnp.float32)
        r = jnp.maximum(a + b1_ref[:, j * _NBL:(j + 1) * _NBL], 0.0)
        scr2[_OFF:_OFF + Mp, _CB + j * _NBL:_CB + (j + 1) * _NBL] = (
            r.astype(jnp.bfloat16))

    for b in range(Bt - 1):
        lo = _OFF + b * S + H
        scr2[lo:lo + (S - H), _CB:_CB + Wl] = jnp.zeros(
            (S - H, Wl), jnp.bfloat16)

    for j in range(NB):
        cw = j * _NBL
        a = jnp.dot(scr2[_OFF - 1:_OFF - 1 + Mp, cw:cw + _KW], w2_ref[0, j],
                    preferred_element_type=jnp.float32)
        a += jnp.dot(scr2[_OFF:_OFF + Mp, cw:cw + _KW], w2_ref[1, j],
                     preferred_element_type=jnp.float32)
        a += jnp.dot(scr2[_OFF + 1:_OFF + 1 + Mp, cw:cw + _KW], w2_ref[2, j],
                     preferred_element_type=jnp.float32)
        outf[:, j * _NBL:(j + 1) * _NBL] = a + b2_ref[:, j * _NBL:(j + 1) * _NBL]

    for b in range(Bt):
        o_ref[b] = outf[b * S:b * S + H, :]


def _band_blocks(w_stack, nb):
    wpad = jnp.pad(w_stack, ((0, 0), (_CB, _CB), (0, 0)))
    blocks = [[wpad[kh, j * _NBL:j * _NBL + _KW, j * _NBL:(j + 1) * _NBL]
               for j in range(nb)] for kh in range(3)]
    return jnp.stack([jnp.stack(row) for row in blocks]).astype(jnp.bfloat16)


@functools.partial(jax.jit, static_argnames=("bt",))
def _resblock(x_lane, w1_stack, b1_lane, w2_stack, b2_lane, *, bt):
    B, H, Wl = x_lane.shape
    NB = Wl // _NBL
    S = -(-(H + 1) // 8) * 8
    Mp = (bt - 1) * S + H

    xb = x_lane.astype(jnp.bfloat16)
    w1b = _band_blocks(w1_stack, NB)
    w2b = _band_blocks(w2_stack, NB)

    body = functools.partial(_rb_kernel, H=H, Bt=bt, S=S, NB=NB)
    return pl.pallas_call(
        body,
        out_shape=jax.ShapeDtypeStruct((B, H, Wl), jnp.float32),
        grid_spec=pltpu.PrefetchScalarGridSpec(
            num_scalar_prefetch=0,
            grid=(B // bt,),
            in_specs=[
                pl.BlockSpec((bt, H, Wl), lambda b: (b, 0, 0)),
                pl.BlockSpec((3, NB, _KW, _NBL), lambda b: (0, 0, 0, 0)),
                pl.BlockSpec((1, Wl), lambda b: (0, 0)),
                pl.BlockSpec((3, NB, _KW, _NBL), lambda b: (0, 0, 0, 0)),
                pl.BlockSpec((1, Wl), lambda b: (0, 0)),
            ],
            out_specs=pl.BlockSpec((bt, H, Wl), lambda b: (b, 0, 0)),
            scratch_shapes=[
                pltpu.VMEM((_OFF + Mp + 1, 2 * _CB + Wl), jnp.bfloat16),
                pltpu.VMEM((_OFF + Mp + 1, 2 * _CB + Wl), jnp.bfloat16),
                pltpu.VMEM((Mp, Wl), jnp.float32),
            ],
        ),
        compiler_params=pltpu.CompilerParams(
            dimension_semantics=("parallel",)),
    )(xb, w1b, b1_lane, w2b, b2_lane)


def kernel(x_lane, w1_stack, b1_lane, w2_stack, b2_lane):
    return _resblock(x_lane, w1_stack, b1_lane, w2_stack, b2_lane, bt=16)


# speedup vs baseline: 1.3286x; 1.3286x over previous
import functools

import jax
import jax.numpy as jnp
from jax.experimental import pallas as pl
from jax.experimental.pallas import tpu as pltpu

_OFF = 8
_CB = 64
_NBL = 128
_KW = 256


def _rb_kernel(x_ref, w1_ref, b1_ref, w2_ref, b2_ref, o_ref,
               scr, scr2, outf, *, H, Bt, S, NB):
    Mp = (Bt - 1) * S + H
    Wl = NB * _NBL

    scr[...] = jnp.zeros_like(scr)
    scr2[...] = jnp.zeros_like(scr2)

    for b in range(Bt):
        lo = _OFF + b * S
        scr[lo:lo + H, _CB:_CB + Wl] = x_ref[b].astype(jnp.bfloat16)

    for j in range(NB):
        cw = j * _NBL
        a = jnp.dot(scr[_OFF - 1:_OFF - 1 + Mp, cw:cw + _KW], w1_ref[0, j],
                    preferred_element_type=jnp.float32)
        a += jnp.dot(scr[_OFF:_OFF + Mp, cw:cw + _KW], w1_ref[1, j],
                     preferred_element_type=jnp.float32)
        a += jnp.dot(scr[_OFF + 1:_OFF + 1 + Mp, cw:cw + _KW], w1_ref[2, j],
                     preferred_element_type=jnp.float32)
        r = jnp.maximum(a + b1_ref[:, j * _NBL:(j + 1) * _NBL], 0.0)
        scr2[_OFF:_OFF + Mp, _CB + j * _NBL:_CB + (j + 1) * _NBL] = (
            r.astype(jnp.bfloat16))

    for b in range(Bt - 1):
        lo = _OFF + b * S + H
        scr2[lo:lo + (S - H), _CB:_CB + Wl] = jnp.zeros(
            (S - H, Wl), jnp.bfloat16)

    for j in range(NB):
        cw = j * _NBL
        a = jnp.dot(scr2[_OFF - 1:_OFF - 1 + Mp, cw:cw + _KW], w2_ref[0, j],
                    preferred_element_type=jnp.float32)
        a += jnp.dot(scr2[_OFF:_OFF + Mp, cw:cw + _KW], w2_ref[1, j],
                     preferred_element_type=jnp.float32)
        a += jnp.dot(scr2[_OFF + 1:_OFF + 1 + Mp, cw:cw + _KW], w2_ref[2, j],
                     preferred_element_type=jnp.float32)
        outf[:, j * _NBL:(j + 1) * _NBL] = a + b2_ref[:, j * _NBL:(j + 1) * _NBL]

    for b in range(Bt):
        o_ref[b] = outf[b * S:b * S + H, :]


def _band_blocks(w_stack, nb):
    Wl = nb * _NBL
    z = jnp.zeros((3, _CB, _NBL), w_stack.dtype)
    blocks = []
    for j in range(nb):
        lo, hi = j * _NBL - _CB, j * _NBL + _KW - _CB
        cols = w_stack[:, :, j * _NBL:(j + 1) * _NBL]
        if lo < 0:
            blk = jnp.concatenate([z, cols[:, 0:hi]], axis=1)
        elif hi > Wl:
            blk = jnp.concatenate([cols[:, lo:Wl], z], axis=1)
        else:
            blk = cols[:, lo:hi]
        blocks.append(blk)
    return jnp.stack(blocks, axis=1).astype(jnp.bfloat16)


@functools.partial(jax.jit, static_argnames=("bt",))
def _resblock(x_lane, w1_stack, b1_lane, w2_stack, b2_lane, *, bt):
    B, H, Wl = x_lane.shape
    NB = Wl // _NBL
    S = -(-(H + 1) // 8) * 8
    Mp = (bt - 1) * S + H

    w1b = _band_blocks(w1_stack, NB)
    w2b = _band_blocks(w2_stack, NB)

    body = functools.partial(_rb_kernel, H=H, Bt=bt, S=S, NB=NB)
    return pl.pallas_call(
        body,
        out_shape=jax.ShapeDtypeStruct((B, H, Wl), jnp.float32),
        grid_spec=pltpu.PrefetchScalarGridSpec(
            num_scalar_prefetch=0,
            grid=(B // bt,),
            in_specs=[
                pl.BlockSpec((bt, H, Wl), lambda b: (b, 0, 0)),
                pl.BlockSpec((3, NB, _KW, _NBL), lambda b: (0, 0, 0, 0)),
                pl.BlockSpec((1, Wl), lambda b: (0, 0)),
                pl.BlockSpec((3, NB, _KW, _NBL), lambda b: (0, 0, 0, 0)),
                pl.BlockSpec((1, Wl), lambda b: (0, 0)),
            ],
            out_specs=pl.BlockSpec((bt, H, Wl), lambda b: (b, 0, 0)),
            scratch_shapes=[
                pltpu.VMEM((_OFF + Mp + 1, 2 * _CB + Wl), jnp.bfloat16),
                pltpu.VMEM((_OFF + Mp + 1, 2 * _CB + Wl), jnp.bfloat16),
                pltpu.VMEM((Mp, Wl), jnp.float32),
            ],
        ),
        compiler_params=pltpu.CompilerParams(
            dimension_semantics=("parallel",)),
    )(x_lane, w1b, b1_lane, w2b, b2_lane)


def kernel(x_lane, w1_stack, b1_lane, w2_stack, b2_lane):
    return _resblock(x_lane, w1_stack, b1_lane, w2_stack, b2_lane, bt=16)


# speedup vs baseline: 1.9017x; 1.4313x over previous
import functools

import jax
import jax.numpy as jnp
from jax.experimental import pallas as pl
from jax.experimental.pallas import tpu as pltpu

_OFF = 8
_CB = 64
_NBL = 128
_KW = 256


def _prep_kernel(*refs, NB):
    j = pl.program_id(0)
    ws, os_ = refs[:8], refs[8:]
    for o_ref, w_refs in zip(os_, (ws[:4], ws[4:])):
        for r in range(4):
            o_ref[:, 0, r * _CB:(r + 1) * _CB, :] = (
                w_refs[r][...].astype(jnp.bfloat16))
        @pl.when(j == 0)
        def _():
            o_ref[:, 0, 0:_CB, :] = jnp.zeros((3, _CB, _NBL), jnp.bfloat16)

        @pl.when(j == NB - 1)
        def _():
            o_ref[:, 0, 3 * _CB:4 * _CB, :] = jnp.zeros(
                (3, _CB, _NBL), jnp.bfloat16)


def _extract_bands(w1_stack, w2_stack, nb):
    def row_spec(r):
        def imap(j):
            rb = 2 * j - 1 + r
            return (0, jnp.clip(rb, 0, 2 * nb - 1), j)
        return pl.BlockSpec((3, _CB, _NBL), imap)

    specs = [row_spec(r) for r in range(4)]
    out_spec = pl.BlockSpec((3, 1, _KW, _NBL), lambda j: (0, j, 0, 0))
    return pl.pallas_call(
        functools.partial(_prep_kernel, NB=nb),
        out_shape=[jax.ShapeDtypeStruct((3, nb, _KW, _NBL), jnp.bfloat16)] * 2,
        grid_spec=pltpu.PrefetchScalarGridSpec(
            num_scalar_prefetch=0,
            grid=(nb,),
            in_specs=specs + specs,
            out_specs=[out_spec, out_spec],
        ),
        compiler_params=pltpu.CompilerParams(
            dimension_semantics=("parallel",)),
    )(w1_stack, w1_stack, w1_stack, w1_stack,
      w2_stack, w2_stack, w2_stack, w2_stack)


def _rb_kernel(x_ref, w1_ref, b1_ref, w2_ref, b2_ref, o_ref,
               scr, scr2, outf, *, H, Bt, S, NB):
    Mp = (Bt - 1) * S + H
    Wl = NB * _NBL

    scr[...] = jnp.zeros_like(scr)
    scr2[...] = jnp.zeros_like(scr2)

    for b in range(Bt):
        lo = _OFF + b * S
        scr[lo:lo + H, _CB:_CB + Wl] = x_ref[b].astype(jnp.bfloat16)

    for j in range(NB):
        cw = j * _NBL
        a = jnp.dot(scr[_OFF - 1:_OFF - 1 + Mp, cw:cw + _KW], w1_ref[0, j],
                    preferred_element_type=jnp.float32)
        a += jnp.dot(scr[_OFF:_OFF + Mp, cw:cw + _KW], w1_ref[1, j],
                     preferred_element_type=jnp.float32)
        a += jnp.dot(scr[_OFF + 1:_OFF + 1 + Mp, cw:cw + _KW], w1_ref[2, j],
                     preferred_element_type=jnp.float32)
        r = jnp.maximum(a + b1_ref[:, j * _NBL:(j + 1) * _NBL], 0.0)
        scr2[_OFF:_OFF + Mp, _CB + j * _NBL:_CB + (j + 1) * _NBL] = (
            r.astype(jnp.bfloat16))

    for b in range(Bt - 1):
        lo = _OFF + b * S + H
        scr2[lo:lo + (S - H), _CB:_CB + Wl] = jnp.zeros(
            (S - H, Wl), jnp.bfloat16)

    for j in range(NB):
        cw = j * _NBL
        a = jnp.dot(scr2[_OFF - 1:_OFF - 1 + Mp, cw:cw + _KW], w2_ref[0, j],
                    preferred_element_type=jnp.float32)
        a += jnp.dot(scr2[_OFF:_OFF + Mp, cw:cw + _KW], w2_ref[1, j],
                     preferred_element_type=jnp.float32)
        a += jnp.dot(scr2[_OFF + 1:_OFF + 1 + Mp, cw:cw + _KW], w2_ref[2, j],
                     preferred_element_type=jnp.float32)
        outf[:, j * _NBL:(j + 1) * _NBL] = a + b2_ref[:, j * _NBL:(j + 1) * _NBL]

    for b in range(Bt):
        o_ref[b] = outf[b * S:b * S + H, :]


@functools.partial(jax.jit, static_argnames=("bt",))
def _resblock(x_lane, w1_stack, b1_lane, w2_stack, b2_lane, *, bt):
    B, H, Wl = x_lane.shape
    NB = Wl // _NBL
    S = -(-(H + 1) // 8) * 8
    Mp = (bt - 1) * S + H

    w1b, w2b = _extract_bands(w1_stack, w2_stack, NB)

    body = functools.partial(_rb_kernel, H=H, Bt=bt, S=S, NB=NB)
    return pl.pallas_call(
        body,
        out_shape=jax.ShapeDtypeStruct((B, H, Wl), jnp.float32),
        grid_spec=pltpu.PrefetchScalarGridSpec(
            num_scalar_prefetch=0,
            grid=(B // bt,),
            in_specs=[
                pl.BlockSpec((bt, H, Wl), lambda b: (b, 0, 0)),
                pl.BlockSpec((3, NB, _KW, _NBL), lambda b: (0, 0, 0, 0)),
                pl.BlockSpec((1, Wl), lambda b: (0, 0)),
                pl.BlockSpec((3, NB, _KW, _NBL), lambda b: (0, 0, 0, 0)),
                pl.BlockSpec((1, Wl), lambda b: (0, 0)),
            ],
            out_specs=pl.BlockSpec((bt, H, Wl), lambda b: (b, 0, 0)),
            scratch_shapes=[
                pltpu.VMEM((_OFF + Mp + 1, 2 * _CB + Wl), jnp.bfloat16),
                pltpu.VMEM((_OFF + Mp + 1, 2 * _CB + Wl), jnp.bfloat16),
                pltpu.VMEM((Mp, Wl), jnp.float32),
            ],
        ),
        compiler_params=pltpu.CompilerParams(
            dimension_semantics=("parallel",)),
    )(x_lane, w1b, b1_lane, w2b, b2_lane)


def kernel(x_lane, w1_stack, b1_lane, w2_stack, b2_lane):
    return _resblock(x_lane, w1_stack, b1_lane, w2_stack, b2_lane, bt=16)


# speedup vs baseline: 2.2537x; 1.1851x over previous
import functools

import jax
import jax.numpy as jnp
from jax.experimental import pallas as pl
from jax.experimental.pallas import tpu as pltpu

_CB = 64
_NBL = 128
_KW = 256


def _prep_kernel(*refs, NB):
    j = pl.program_id(0)
    ws, os_ = refs[:8], refs[8:]
    for o_ref, w_refs in zip(os_, (ws[:4], ws[4:])):
        for r in range(4):
            o_ref[:, 0, r * _CB:(r + 1) * _CB, :] = (
                w_refs[r][...].astype(jnp.bfloat16))
        @pl.when(j == 0)
        def _():
            o_ref[:, 0, 0:_CB, :] = jnp.zeros((3, _CB, _NBL), jnp.bfloat16)

        @pl.when(j == NB - 1)
        def _():
            o_ref[:, 0, 3 * _CB:4 * _CB, :] = jnp.zeros(
                (3, _CB, _NBL), jnp.bfloat16)


def _extract_bands(w1_stack, w2_stack, nb):
    def row_spec(r):
        def imap(j):
            rb = 2 * j - 1 + r
            return (0, jnp.clip(rb, 0, 2 * nb - 1), j)
        return pl.BlockSpec((3, _CB, _NBL), imap)

    specs = [row_spec(r) for r in range(4)]
    out_spec = pl.BlockSpec((3, 1, _KW, _NBL), lambda j: (0, j, 0, 0))
    return pl.pallas_call(
        functools.partial(_prep_kernel, NB=nb),
        out_shape=[jax.ShapeDtypeStruct((3, nb, _KW, _NBL), jnp.bfloat16)] * 2,
        grid_spec=pltpu.PrefetchScalarGridSpec(
            num_scalar_prefetch=0,
            grid=(nb,),
            in_specs=specs + specs,
            out_specs=[out_spec, out_spec],
        ),
        compiler_params=pltpu.CompilerParams(
            dimension_semantics=("parallel",)),
    )(w1_stack, w1_stack, w1_stack, w1_stack,
      w2_stack, w2_stack, w2_stack, w2_stack)


def _rb_kernel(x_ref, w1_ref, b1_ref, w2_ref, b2_ref, o_ref,
               xA, xB, xC, rA, rB, rC, *, H, Bt, NB):
    M = Bt * H
    Wl = NB * _NBL

    @pl.when(pl.program_id(0) == 0)
    def _():
        for s in (xA, xB, xC, rA, rB, rC):
            s[...] = jnp.zeros_like(s)

    for b in range(Bt):
        xv = x_ref[b].astype(jnp.bfloat16)
        lo = b * H
        xB[lo:lo + H, _CB:_CB + Wl] = xv
        xA[lo + 1:lo + H, _CB:_CB + Wl] = xv[0:H - 1]
        xC[lo:lo + H - 1, _CB:_CB + Wl] = xv[1:H]

    for j in range(NB):
        cw = j * _NBL
        a = jnp.dot(xA[:, cw:cw + _KW], w1_ref[0, j],
                    preferred_element_type=jnp.float32)
        a += jnp.dot(xB[:, cw:cw + _KW], w1_ref[1, j],
                     preferred_element_type=jnp.float32)
        a += jnp.dot(xC[:, cw:cw + _KW], w1_ref[2, j],
                     preferred_element_type=jnp.float32)
        r = jnp.maximum(a + b1_ref[:, cw:cw + _NBL], 0.0).astype(jnp.bfloat16)
        r3 = r.reshape(Bt, H, _NBL)
        zrow = jnp.zeros((Bt, 1, _NBL), jnp.bfloat16)
        co = _CB + cw
        rB[:, co:co + _NBL] = r
        rA[:, co:co + _NBL] = jnp.concatenate(
            [zrow, r3[:, 0:H - 1]], axis=1).reshape(M, _NBL)
        rC[:, co:co + _NBL] = jnp.concatenate(
            [r3[:, 1:H], zrow], axis=1).reshape(M, _NBL)

    for j in range(NB):
        cw = j * _NBL
        a = jnp.dot(rA[:, cw:cw + _KW], w2_ref[0, j],
                    preferred_element_type=jnp.float32)
        a += jnp.dot(rB[:, cw:cw + _KW], w2_ref[1, j],
                     preferred_element_type=jnp.float32)
        a += jnp.dot(rC[:, cw:cw + _KW], w2_ref[2, j],
                     preferred_element_type=jnp.float32)
        a3 = (a + b2_ref[:, cw:cw + _NBL]).reshape(Bt, H, _NBL)
        o_ref[:, :, cw:cw + _NBL] = a3


@functools.partial(jax.jit, static_argnames=("bt",))
def _resblock(x_lane, w1_stack, b1_lane, w2_stack, b2_lane, *, bt):
    B, H, Wl = x_lane.shape
    NB = Wl // _NBL
    M = bt * H

    w1b, w2b = _extract_bands(w1_stack, w2_stack, NB)

    body = functools.partial(_rb_kernel, H=H, Bt=bt, NB=NB)
    scr = pltpu.VMEM((M, 2 * _CB + Wl), jnp.bfloat16)
    return pl.pallas_call(
        body,
        out_shape=jax.ShapeDtypeStruct((B, H, Wl), jnp.float32),
        grid_spec=pltpu.PrefetchScalarGridSpec(
            num_scalar_prefetch=0,
            grid=(B // bt,),
            in_specs=[
                pl.BlockSpec((bt, H, Wl), lambda b: (b, 0, 0)),
                pl.BlockSpec((3, NB, _KW, _NBL), lambda b: (0, 0, 0, 0)),
                pl.BlockSpec((1, Wl), lambda b: (0, 0)),
                pl.BlockSpec((3, NB, _KW, _NBL), lambda b: (0, 0, 0, 0)),
                pl.BlockSpec((1, Wl), lambda b: (0, 0)),
            ],
            out_specs=pl.BlockSpec((bt, H, Wl), lambda b: (b, 0, 0)),
            scratch_shapes=[scr] * 6,
        ),
        compiler_params=pltpu.CompilerParams(
            dimension_semantics=("arbitrary",)),
    )(x_lane, w1b, b1_lane, w2b, b2_lane)


def kernel(x_lane, w1_stack, b1_lane, w2_stack, b2_lane):
    return _resblock(x_lane, w1_stack, b1_lane, w2_stack, b2_lane, bt=16)


# speedup vs baseline: 2.2662x; 1.0056x over previous
import functools

import jax
import jax.numpy as jnp
from jax.experimental import pallas as pl
from jax.experimental.pallas import tpu as pltpu

_CB = 64
_NBL = 128
_KW = 256


def _prep_kernel(*refs, NB):
    j = pl.program_id(0)
    ws, os_ = refs[:8], refs[8:]
    for o_ref, w_refs in zip(os_, (ws[:4], ws[4:])):
        for r in range(4):
            o_ref[:, 0, r * _CB:(r + 1) * _CB, :] = (
                w_refs[r][...].astype(jnp.bfloat16))
        @pl.when(j == 0)
        def _():
            o_ref[:, 0, 0:_CB, :] = jnp.zeros((3, _CB, _NBL), jnp.bfloat16)

        @pl.when(j == NB - 1)
        def _():
            o_ref[:, 0, 3 * _CB:4 * _CB, :] = jnp.zeros(
                (3, _CB, _NBL), jnp.bfloat16)


def _extract_bands(w1_stack, w2_stack, nb):
    def row_spec(r):
        def imap(j):
            rb = 2 * j - 1 + r
            return (0, jnp.clip(rb, 0, 2 * nb - 1), j)
        return pl.BlockSpec((3, _CB, _NBL), imap)

    specs = [row_spec(r) for r in range(4)]
    out_spec = pl.BlockSpec((3, 1, _KW, _NBL), lambda j: (0, j, 0, 0))
    return pl.pallas_call(
        functools.partial(_prep_kernel, NB=nb),
        out_shape=[jax.ShapeDtypeStruct((3, nb, _KW, _NBL), jnp.bfloat16)] * 2,
        grid_spec=pltpu.PrefetchScalarGridSpec(
            num_scalar_prefetch=0,
            grid=(nb,),
            in_specs=specs + specs,
            out_specs=[out_spec, out_spec],
        ),
        compiler_params=pltpu.CompilerParams(
            dimension_semantics=("parallel",)),
    )(w1_stack, w1_stack, w1_stack, w1_stack,
      w2_stack, w2_stack, w2_stack, w2_stack)


def _rb_kernel(x_ref, w1_ref, b1_ref, w2_ref, b2_ref, o_ref,
               xA, xB, xC, rA, rB, rC, *, H, Bt, NB):
    M = Bt * H
    Wl = NB * _NBL

    @pl.when(pl.program_id(0) == 0)
    def _():
        for s in (xA, xB, xC, rA, rB, rC):
            s[...] = jnp.zeros_like(s)

    for b in range(Bt):
        xv = x_ref[b].astype(jnp.bfloat16)
        lo = b * H
        xB[lo:lo + H, _CB:_CB + Wl] = xv
        xA[lo + 1:lo + H, _CB:_CB + Wl] = xv[0:H - 1]
        xC[lo:lo + H - 1, _CB:_CB + Wl] = xv[1:H]

    for j in range(NB):
        cw = j * _NBL
        a = jnp.dot(xA[:, cw:cw + _KW], w1_ref[0, j],
                    preferred_element_type=jnp.float32)
        a += jnp.dot(xB[:, cw:cw + _KW], w1_ref[1, j],
                     preferred_element_type=jnp.float32)
        a += jnp.dot(xC[:, cw:cw + _KW], w1_ref[2, j],
                     preferred_element_type=jnp.float32)
        r = jnp.maximum(a + b1_ref[:, cw:cw + _NBL], 0.0).astype(jnp.bfloat16)
        r3 = r.reshape(Bt, H, _NBL)
        zrow = jnp.zeros((Bt, 1, _NBL), jnp.bfloat16)
        co = _CB + cw
        rB[:, co:co + _NBL] = r
        rA[:, co:co + _NBL] = jnp.concatenate(
            [zrow, r3[:, 0:H - 1]], axis=1).reshape(M, _NBL)
        rC[:, co:co + _NBL] = jnp.concatenate(
            [r3[:, 1:H], zrow], axis=1).reshape(M, _NBL)

    for j in range(NB):
        cw = j * _NBL
        a = jnp.dot(rA[:, cw:cw + _KW], w2_ref[0, j],
                    preferred_element_type=jnp.float32)
        a += jnp.dot(rB[:, cw:cw + _KW], w2_ref[1, j],
                     preferred_element_type=jnp.float32)
        a += jnp.dot(rC[:, cw:cw + _KW], w2_ref[2, j],
                     preferred_element_type=jnp.float32)
        a3 = (a + b2_ref[:, cw:cw + _NBL]).reshape(Bt, H, _NBL)
        o_ref[:, :, cw:cw + _NBL] = a3


@functools.partial(jax.jit, static_argnames=("bt",))
def _resblock(x_lane, w1_stack, b1_lane, w2_stack, b2_lane, *, bt):
    B, H, Wl = x_lane.shape
    NB = Wl // _NBL
    M = bt * H

    w1b, w2b = _extract_bands(w1_stack, w2_stack, NB)

    body = functools.partial(_rb_kernel, H=H, Bt=bt, NB=NB)
    scr = pltpu.VMEM((M, 2 * _CB + Wl), jnp.bfloat16)
    return pl.pallas_call(
        body,
        out_shape=jax.ShapeDtypeStruct((B, H, Wl), jnp.float32),
        grid_spec=pltpu.PrefetchScalarGridSpec(
            num_scalar_prefetch=0,
            grid=(B // bt,),
            in_specs=[
                pl.BlockSpec((bt, H, Wl), lambda b: (b, 0, 0)),
                pl.BlockSpec((3, NB, _KW, _NBL), lambda b: (0, 0, 0, 0)),
                pl.BlockSpec((1, Wl), lambda b: (0, 0)),
                pl.BlockSpec((3, NB, _KW, _NBL), lambda b: (0, 0, 0, 0)),
                pl.BlockSpec((1, Wl), lambda b: (0, 0)),
            ],
            out_specs=pl.BlockSpec((bt, H, Wl), lambda b: (b, 0, 0)),
            scratch_shapes=[scr] * 6,
        ),
        compiler_params=pltpu.CompilerParams(
            dimension_semantics=("arbitrary",)),
    )(x_lane, w1b, b1_lane, w2b, b2_lane)


def kernel(x_lane, w1_stack, b1_lane, w2_stack, b2_lane):
    return _resblock(x_lane, w1_stack, b1_lane, w2_stack, b2_lane, bt=32)


# speedup vs baseline: 2.7180x; 1.1994x over previous
import functools

import jax
import jax.numpy as jnp
from jax.experimental import pallas as pl
from jax.experimental.pallas import tpu as pltpu

_CB = 64
_NBL = 128
_KW = 256


def _rb_kernel(*args, H, Bt, NB):
    nv = 4 * NB
    x_ref = args[0]
    w1v = args[1:1 + nv]
    b1_ref = args[1 + nv]
    w2v = args[2 + nv:2 + 2 * nv]
    b2_ref = args[2 + 2 * nv]
    o_ref = args[3 + 2 * nv]
    xA, xB, xC, rA, rB, rC, w1s, w2s = args[4 + 2 * nv:]

    M = Bt * H
    Wl = NB * _NBL

    @pl.when(pl.program_id(0) == 0)
    def _():
        for s in (xA, xB, xC, rA, rB, rC):
            s[...] = jnp.zeros_like(s)
        for ws, wv in ((w1s, w1v), (w2s, w2v)):
            for j in range(NB):
                for r in range(4):
                    rb = 2 * j - 1 + r
                    if 0 <= rb < 2 * NB:
                        ws[j, :, r * _CB:(r + 1) * _CB, :] = (
                            wv[4 * j + r][...].astype(jnp.bfloat16))
                    else:
                        ws[j, :, r * _CB:(r + 1) * _CB, :] = jnp.zeros(
                            (3, _CB, _NBL), jnp.bfloat16)

    for b in range(Bt):
        xv = x_ref[b].astype(jnp.bfloat16)
        lo = b * H
        xB[lo:lo + H, _CB:_CB + Wl] = xv
        xA[lo + 1:lo + H, _CB:_CB + Wl] = xv[0:H - 1]
        xC[lo:lo + H - 1, _CB:_CB + Wl] = xv[1:H]

    for j in range(NB):
        cw = j * _NBL
        a = jnp.dot(xA[:, cw:cw + _KW], w1s[j, 0],
                    preferred_element_type=jnp.float32)
        a += jnp.dot(xB[:, cw:cw + _KW], w1s[j, 1],
                     preferred_element_type=jnp.float32)
        a += jnp.dot(xC[:, cw:cw + _KW], w1s[j, 2],
                     preferred_element_type=jnp.float32)
        r = jnp.maximum(a + b1_ref[:, cw:cw + _NBL], 0.0).astype(jnp.bfloat16)
        r3 = r.reshape(Bt, H, _NBL)
        zrow = jnp.zeros((Bt, 1, _NBL), jnp.bfloat16)
        co = _CB + cw
        rB[:, co:co + _NBL] = r
        rA[:, co:co + _NBL] = jnp.concatenate(
            [zrow, r3[:, 0:H - 1]], axis=1).reshape(M, _NBL)
        rC[:, co:co + _NBL] = jnp.concatenate(
            [r3[:, 1:H], zrow], axis=1).reshape(M, _NBL)

    for j in range(NB):
        cw = j * _NBL
        a = jnp.dot(rA[:, cw:cw + _KW], w2s[j, 0],
                    preferred_element_type=jnp.float32)
        a += jnp.dot(rB[:, cw:cw + _KW], w2s[j, 1],
                     preferred_element_type=jnp.float32)
        a += jnp.dot(rC[:, cw:cw + _KW], w2s[j, 2],
                     preferred_element_type=jnp.float32)
        a3 = (a + b2_ref[:, cw:cw + _NBL]).reshape(Bt, H, _NBL)
        o_ref[:, :, cw:cw + _NBL] = a3


@functools.partial(jax.jit, static_argnames=("bt",))
def _resblock(x_lane, w1_stack, b1_lane, w2_stack, b2_lane, *, bt):
    B, H, Wl = x_lane.shape
    NB = Wl // _NBL
    M = bt * H

    def view_spec(j, r):
        rb = min(max(2 * j - 1 + r, 0), 2 * NB - 1)
        return pl.BlockSpec((3, _CB, _NBL), lambda b, rb=rb, j=j: (0, rb, j))

    vspecs = [view_spec(j, r) for j in range(NB) for r in range(4)]
    body = functools.partial(_rb_kernel, H=H, Bt=bt, NB=NB)
    scr = pltpu.VMEM((M, 2 * _CB + Wl), jnp.bfloat16)
    wscr = pltpu.VMEM((NB, 3, _KW, _NBL), jnp.bfloat16)
    return pl.pallas_call(
        body,
        out_shape=jax.ShapeDtypeStruct((B, H, Wl), jnp.float32),
        grid_spec=pltpu.PrefetchScalarGridSpec(
            num_scalar_prefetch=0,
            grid=(B // bt,),
            in_specs=(
                [pl.BlockSpec((bt, H, Wl), lambda b: (b, 0, 0))]
                + vspecs
                + [pl.BlockSpec((1, Wl), lambda b: (0, 0))]
                + vspecs
                + [pl.BlockSpec((1, Wl), lambda b: (0, 0))]
            ),
            out_specs=pl.BlockSpec((bt, H, Wl), lambda b: (b, 0, 0)),
            scratch_shapes=[scr] * 6 + [wscr] * 2,
        ),
        compiler_params=pltpu.CompilerParams(
            dimension_semantics=("arbitrary",)),
    )(x_lane, *([w1_stack] * (4 * NB)), b1_lane,
      *([w2_stack] * (4 * NB)), b2_lane)


def kernel(x_lane, w1_stack, b1_lane, w2_stack, b2_lane):
    return _resblock(x_lane, w1_stack, b1_lane, w2_stack, b2_lane, bt=32)
